# Initial kernel scaffold; baseline (speedup 1.0000x reference)
import jax
import jax.numpy as jnp
from jax.experimental import pallas as pl


def kernel(x, edge_index, edge_type, W, h_bias):
    raise NotImplementedError("write your pallas kernel here")


# speedup vs baseline: 15.1217x; 15.1217x over previous
import functools

import jax
import jax.numpy as jnp
from jax import lax
from jax.experimental import pallas as pl
from jax.experimental.pallas import tpu as pltpu
from jax.experimental.pallas import tpu_sc as plsc

N_NODES = 10000
N_EDGES = 320000
D = 128
N_REL = 8

NTILES = 32
CH = 128
EPT = 10240
NCH = EPT // CH
E_PAD = NTILES * EPT
NBINS = 81920
BPT = NBINS // 16
N_ROWS = 10240
RPT = N_ROWS // 16

_mesh = plsc.VectorSubcoreMesh(core_axis_name="c", subcore_axis_name="s")


def _mm_body(x_ref, w_ref, o_ref):
    o_ref[0] = jnp.dot(x_ref[...], w_ref[0],
                       preferred_element_type=jnp.float32)


def _xw_table(x, W):
    M_BLK = 2000
    xw = pl.pallas_call(
        _mm_body,
        grid=(N_REL, N_NODES // M_BLK),
        in_specs=[
            pl.BlockSpec((M_BLK, D), lambda r, m: (m, 0)),
            pl.BlockSpec((1, D, D), lambda r, m: (r, 0, 0)),
        ],
        out_specs=pl.BlockSpec((1, M_BLK, D), lambda r, m: (r, m, 0)),
        out_shape=jax.ShapeDtypeStruct((N_REL, N_NODES, D), jnp.float32),
    )(x, W)
    return xw.reshape(N_REL * N_NODES, D)


@functools.partial(
    pl.kernel,
    mesh=_mesh,
    out_type=jax.ShapeDtypeStruct((2 * NBINS,), jnp.float32),
    scratch_types=[
        pltpu.VMEM((CH,), jnp.int32),
        pltpu.VMEM((CH,), jnp.int32),
        pltpu.VMEM((1, CH), jnp.int32),
        pltpu.VMEM((CH,), jnp.float32),
        pltpu.VMEM((BPT,), jnp.float32),
        pltpu.VMEM_SHARED((NBINS,), jnp.float32),
    ],
)
def _deg_kernel(dst_hbm, et_hbm, out_hbm, dst_v, et_v, key_v, ones_v,
                bounce_v, hist_sh):
    cid = lax.axis_index("c")
    sid = lax.axis_index("s")
    wid = cid * 16 + sid

    def _zero16(i, _):
        bounce_v[pl.ds(i * 16, 16)] = jnp.zeros((16,), jnp.float32)
        return 0

    lax.fori_loop(0, BPT // 16, _zero16, 0)

    def _ones16(i, _):
        ones_v[pl.ds(i * 16, 16)] = jnp.ones((16,), jnp.float32)
        return 0

    lax.fori_loop(0, CH // 16, _ones16, 0)

    pltpu.sync_copy(bounce_v, hist_sh.at[pl.ds(sid * BPT, BPT)])
    plsc.subcore_barrier()

    base = wid * EPT

    def _chunk(j, _):
        off = base + j * CH
        pltpu.sync_copy(dst_hbm.at[pl.ds(off, CH)], dst_v)
        pltpu.sync_copy(et_hbm.at[pl.ds(off, CH)], et_v)

        def _keys(i, _):
            t = et_v[pl.ds(i * 16, 16)]
            d = dst_v[pl.ds(i * 16, 16)]
            key_v[0, pl.ds(i * 16, 16)] = t * N_NODES + d
            return 0

        lax.fori_loop(0, CH // 16, _keys, 0)
        pltpu.sync_copy(ones_v, hist_sh.at[key_v.at[0]], add=True)
        return 0

    lax.fori_loop(0, NCH, _chunk, 0)
    plsc.subcore_barrier()

    pltpu.sync_copy(hist_sh.at[pl.ds(sid * BPT, BPT)], bounce_v)
    pltpu.sync_copy(bounce_v, out_hbm.at[pl.ds(cid * NBINS + sid * BPT, BPT)])


def _inv_body(p_ref, o_ref):
    s = p_ref[0] + p_ref[1]
    row = lax.broadcasted_iota(jnp.int32, (NBINS // 128, 128), 0)
    col = lax.broadcasted_iota(jnp.int32, (NBINS // 128, 128), 1)
    real = (row * 128 + col) < N_REL * N_NODES
    o_ref[...] = jnp.where(real, 1.0 / jnp.maximum(s, 1.0), 0.0)


def _inv_deg(parts):
    inv = pl.pallas_call(
        _inv_body,
        out_shape=jax.ShapeDtypeStruct((NBINS // 128, 128), jnp.float32),
    )(parts.reshape(2, NBINS // 128, 128))
    return inv.reshape(NBINS)


@functools.partial(
    pl.kernel,
    mesh=_mesh,
    out_type=jax.ShapeDtypeStruct((2, N_ROWS, D), jnp.float32),
    scratch_types=[
        pltpu.VMEM((CH,), jnp.int32),
        pltpu.VMEM((CH,), jnp.int32),
        pltpu.VMEM((CH,), jnp.int32),
        pltpu.VMEM((1, CH), jnp.int32),
        pltpu.VMEM((1, CH), jnp.int32),
        pltpu.VMEM((1, CH), jnp.int32),
        pltpu.VMEM((CH,), jnp.float32),
        pltpu.VMEM((CH, D), jnp.float32),
        pltpu.VMEM_SHARED((N_ROWS, D), jnp.float32),
    ],
)
def _edge_kernel(xw_hbm, src_hbm, dst_hbm, et_hbm, inv_hbm, out_hbm,
                 src_v, dst_v, et_v, kg_v, di_v, ks_v, s_v, rows_v, acc_sh):
    cid = lax.axis_index("c")
    sid = lax.axis_index("s")
    wid = cid * 16 + sid

    def _zrow(i, _):
        for q in range(D // 16):
            rows_v[i, pl.ds(q * 16, 16)] = jnp.zeros((16,), jnp.float32)
        return 0

    lax.fori_loop(0, CH, _zrow, 0)

    rbase = sid * RPT
    for k in range(RPT // CH):
        pltpu.sync_copy(rows_v, acc_sh.at[pl.ds(rbase + k * CH, CH)])
    plsc.subcore_barrier()

    base = wid * EPT

    def _chunk(j, _):
        off = base + j * CH
        pltpu.sync_copy(src_hbm.at[pl.ds(off, CH)], src_v)
        pltpu.sync_copy(dst_hbm.at[pl.ds(off, CH)], dst_v)
        pltpu.sync_copy(et_hbm.at[pl.ds(off, CH)], et_v)

        def _keys(i, _):
            sl = pl.ds(i * 16, 16)
            t = et_v[sl]
            s16 = src_v[sl]
            d16 = dst_v[sl]
            valid = t < N_REL
            kg_v[0, sl] = jnp.where(valid, t * N_NODES + s16, 0)
            di_v[0, sl] = d16
            ks_v[0, sl] = t * N_NODES + d16
            return 0

        lax.fori_loop(0, CH // 16, _keys, 0)

        pltpu.sync_copy(inv_hbm.at[ks_v.at[0]], s_v)
        pltpu.sync_copy(xw_hbm.at[kg_v.at[0]], rows_v)

        def _scale(g, _):
            s16 = s_v[pl.ds(g * 16, 16)]
            for i16 in range(16):
                sc = lax.gather(
                    s16, jnp.full((16, 1), i16, dtype=jnp.int32),
                    lax.GatherDimensionNumbers(
                        offset_dims=(), collapsed_slice_dims=(0,),
                        start_index_map=(0,)),
                    (1,), mode=lax.GatherScatterMode.PROMISE_IN_BOUNDS)
                row = g * 16 + i16
                for q in range(D // 16):
                    sl = pl.ds(q * 16, 16)
                    rows_v[row, sl] = rows_v[row, sl] * sc
            return 0

        lax.fori_loop(0, CH // 16, _scale, 0)

        pltpu.sync_copy(rows_v, acc_sh.at[di_v.at[0]], add=True)
        return 0

    lax.fori_loop(0, NCH, _chunk, 0)
    plsc.subcore_barrier()

    for k in range(RPT // CH):
        pltpu.sync_copy(acc_sh.at[pl.ds(rbase + k * CH, CH)], rows_v)
        pltpu.sync_copy(rows_v, out_hbm.at[cid, pl.ds(rbase + k * CH, CH)])


def _fin_body(p_ref, b_ref, o_ref):
    o_ref[...] = p_ref[0] + p_ref[1] + b_ref[...]


def _combine(parts, h_bias):
    return pl.pallas_call(
        _fin_body,
        out_shape=jax.ShapeDtypeStruct((N_NODES, D), jnp.float32),
    )(parts, h_bias.reshape(1, D))


def kernel(x, edge_index, edge_type, W, h_bias):
    src = edge_index[0].astype(jnp.int32)
    dst = edge_index[1].astype(jnp.int32)
    et = edge_type.astype(jnp.int32)

    pad = E_PAD - N_EDGES
    src_p = jnp.pad(src, (0, pad))
    dst_p = jnp.pad(dst, (0, pad))
    et_p = jnp.pad(et, (0, pad), constant_values=N_REL)

    xw = _xw_table(x, W)
    deg_parts = _deg_kernel(dst_p, et_p)
    inv = _inv_deg(deg_parts)
    parts = _edge_kernel(xw, src_p, dst_p, et_p, inv)[:, :N_NODES, :]
    return _combine(parts, h_bias)


# speedup vs baseline: 24.9022x; 1.6468x over previous
import functools

import jax
import jax.numpy as jnp
from jax import lax
from jax.experimental import pallas as pl
from jax.experimental.pallas import tpu as pltpu
from jax.experimental.pallas import tpu_sc as plsc

N_NODES = 10000
N_EDGES = 320000
D = 128
N_REL = 8

NTILES = 32
CH = 128
EPT = 10240
NCH = EPT // CH
E_PAD = NTILES * EPT
NBINS = 81920
BPT = NBINS // 16
N_ROWS = 10240
RPT = N_ROWS // 16

_mesh = plsc.VectorSubcoreMesh(core_axis_name="c", subcore_axis_name="s")


def _mm_body(x_ref, w_ref, o_ref):
    o_ref[0] = jnp.dot(x_ref[...], w_ref[0],
                       preferred_element_type=jnp.float32)


def _xw_table(x, W):
    M_BLK = 2000
    xw = pl.pallas_call(
        _mm_body,
        grid=(N_REL, N_NODES // M_BLK),
        in_specs=[
            pl.BlockSpec((M_BLK, D), lambda r, m: (m, 0)),
            pl.BlockSpec((1, D, D), lambda r, m: (r, 0, 0)),
        ],
        out_specs=pl.BlockSpec((1, M_BLK, D), lambda r, m: (r, m, 0)),
        out_shape=jax.ShapeDtypeStruct((N_REL, N_NODES, D), jnp.float32),
    )(x, W)
    return xw.reshape(N_REL * N_NODES, D)


@functools.partial(
    pl.kernel,
    mesh=_mesh,
    out_type=jax.ShapeDtypeStruct((2 * NBINS,), jnp.float32),
    scratch_types=[
        pltpu.VMEM((EPT,), jnp.int32),
        pltpu.VMEM((1, CH), jnp.int32),
        pltpu.VMEM((1, CH), jnp.int32),
        pltpu.VMEM((CH,), jnp.float32),
        pltpu.VMEM((BPT,), jnp.float32),
        pltpu.VMEM_SHARED((NBINS,), jnp.float32),
        pltpu.SemaphoreType.DMA,
        pltpu.SemaphoreType.DMA,
    ],
)
def _deg_kernel(pk_hbm, out_hbm, pk_v, k0_v, k1_v, ones_v,
                bounce_v, hist_sh, sem0, sem1):
    cid = lax.axis_index("c")
    sid = lax.axis_index("s")
    wid = cid * 16 + sid

    def _zero16(i, _):
        bounce_v[pl.ds(i * 16, 16)] = jnp.zeros((16,), jnp.float32)
        return 0

    lax.fori_loop(0, BPT // 16, _zero16, 0)

    def _ones16(i, _):
        ones_v[pl.ds(i * 16, 16)] = jnp.ones((16,), jnp.float32)
        return 0

    lax.fori_loop(0, CH // 16, _ones16, 0)

    pltpu.sync_copy(pk_hbm.at[pl.ds(wid * EPT, EPT)], pk_v)
    pltpu.sync_copy(bounce_v, hist_sh.at[pl.ds(sid * BPT, BPT)])
    plsc.subcore_barrier()

    def _keys(j, kv):
        def _k16(i, _):
            p = pk_v[pl.ds(j * CH + i * 16, 16)]
            t = lax.shift_right_logical(p, 28)
            d = jnp.bitwise_and(lax.shift_right_logical(p, 14), 16383)
            kv[0, pl.ds(i * 16, 16)] = t * N_NODES + d
            return 0
        lax.fori_loop(0, CH // 16, _k16, 0)

    def _fire(kv, sem):
        pltpu.async_copy(ones_v, hist_sh.at[kv.at[0]], sem, add=True)

    def _drain(sem):
        pltpu.make_async_copy(ones_v, hist_sh.at[k0_v.at[0]], sem).wait()

    _keys(0, k0_v)
    _fire(k0_v, sem0)
    _keys(1, k1_v)
    _fire(k1_v, sem1)

    def _pair(g2, _):
        g = g2 * 2
        _drain(sem0)
        _keys(g + 2, k0_v)
        _fire(k0_v, sem0)
        _drain(sem1)
        _keys(g + 3, k1_v)
        _fire(k1_v, sem1)
        return 0

    lax.fori_loop(0, (NCH - 2) // 2, _pair, 0)
    _drain(sem0)
    _drain(sem1)
    plsc.subcore_barrier()

    pltpu.sync_copy(hist_sh.at[pl.ds(sid * BPT, BPT)], bounce_v)
    pltpu.sync_copy(bounce_v, out_hbm.at[pl.ds(cid * NBINS + sid * BPT, BPT)])


def _inv_body(p_ref, o_ref):
    s = p_ref[0] + p_ref[1]
    row = lax.broadcasted_iota(jnp.int32, (NBINS // 128, 128), 0)
    col = lax.broadcasted_iota(jnp.int32, (NBINS // 128, 128), 1)
    real = (row * 128 + col) < N_REL * N_NODES
    o_ref[...] = jnp.where(real, 1.0 / jnp.maximum(s, 1.0), 0.0)


def _inv_deg(parts):
    inv = pl.pallas_call(
        _inv_body,
        out_shape=jax.ShapeDtypeStruct((NBINS // 128, 128), jnp.float32),
    )(parts.reshape(2, NBINS // 128, 128))
    return inv.reshape(NBINS)


@functools.partial(
    pl.kernel,
    mesh=_mesh,
    out_type=jax.ShapeDtypeStruct((2, N_ROWS, D), jnp.float32),
    scratch_types=[
        pltpu.VMEM((CH,), jnp.int32),
        pltpu.VMEM((CH,), jnp.int32),
        pltpu.VMEM((1, CH), jnp.int32),
        pltpu.VMEM((1, CH), jnp.int32),
        pltpu.VMEM((1, CH), jnp.int32),
        pltpu.VMEM((1, CH), jnp.int32),
        pltpu.VMEM((1, CH), jnp.int32),
        pltpu.VMEM((1, CH), jnp.int32),
        pltpu.VMEM((CH,), jnp.float32),
        pltpu.VMEM((CH,), jnp.float32),
        pltpu.VMEM((CH, D), jnp.float32),
        pltpu.VMEM((CH, D), jnp.float32),
        pltpu.VMEM_SHARED((N_ROWS, D), jnp.float32),
        pltpu.SemaphoreType.DMA,
        pltpu.SemaphoreType.DMA,
        pltpu.SemaphoreType.DMA,
        pltpu.SemaphoreType.DMA,
    ],
)
def _edge_kernel(xw_hbm, pk_hbm, inv_hbm, out_hbm,
                 pk0_v, pk1_v, kg0_v, kg1_v, di0_v, di1_v, ks0_v, ks1_v,
                 s0_v, s1_v, rows0_v, rows1_v, acc_sh,
                 semi0, semi1, semg0, semg1):
    cid = lax.axis_index("c")
    sid = lax.axis_index("s")
    wid = cid * 16 + sid

    def _zrow(i, _):
        for q in range(D // 16):
            rows0_v[i, pl.ds(q * 16, 16)] = jnp.zeros((16,), jnp.float32)
        return 0

    lax.fori_loop(0, CH, _zrow, 0)

    rbase = sid * RPT
    for k in range(RPT // CH):
        pltpu.sync_copy(rows0_v, acc_sh.at[pl.ds(rbase + k * CH, CH)])
    plsc.subcore_barrier()

    base = wid * EPT

    def _fire_idx(j, pk_v, sem):
        pltpu.async_copy(pk_hbm.at[pl.ds(base + j * CH, CH)], pk_v, sem)

    def _prep(pk_v, kg_v, di_v, ks_v, s_v, rows_v, semi, semg):
        pltpu.make_async_copy(pk_hbm.at[pl.ds(0, CH)], pk_v, semi).wait()

        def _k16(i, _):
            sl = pl.ds(i * 16, 16)
            p = pk_v[sl]
            t = lax.shift_right_logical(p, 28)
            d = jnp.bitwise_and(lax.shift_right_logical(p, 14), 16383)
            s16 = jnp.bitwise_and(p, 16383)
            valid = t < N_REL
            kg_v[0, sl] = jnp.where(valid, t * N_NODES + s16, 0)
            di_v[0, sl] = d
            ks_v[0, sl] = t * N_NODES + d
            return 0

        lax.fori_loop(0, CH // 16, _k16, 0)
        pltpu.async_copy(inv_hbm.at[ks_v.at[0]], s_v, semg)
        pltpu.async_copy(xw_hbm.at[kg_v.at[0]], rows_v, semg)

    def _finish(di_v, s_v, rows_v, semg):
        pltpu.make_async_copy(inv_hbm.at[pl.ds(0, CH)], s_v, semg).wait()
        pltpu.make_async_copy(xw_hbm.at[pl.ds(0, CH)], rows_v, semg).wait()

        def _sg(g, _):
            s16 = s_v[pl.ds(g * 16, 16)]
            for i16 in range(16):
                sc = lax.gather(
                    s16, jnp.full((16, 1), i16, dtype=jnp.int32),
                    lax.GatherDimensionNumbers(
                        offset_dims=(), collapsed_slice_dims=(0,),
                        start_index_map=(0,)),
                    (1,), mode=lax.GatherScatterMode.PROMISE_IN_BOUNDS)
                row = g * 16 + i16
                for q in range(D // 16):
                    sl = pl.ds(q * 16, 16)
                    rows_v[row, sl] = rows_v[row, sl] * sc
            return 0

        lax.fori_loop(0, CH // 16, _sg, 0)
        pltpu.sync_copy(rows_v, acc_sh.at[di_v.at[0]], add=True)

    _fire_idx(0, pk0_v, semi0)
    _fire_idx(1, pk1_v, semi1)
    _prep(pk0_v, kg0_v, di0_v, ks0_v, s0_v, rows0_v, semi0, semg0)

    def _pair(g2, _):
        g = g2 * 2
        _fire_idx(g + 2, pk0_v, semi0)
        _prep(pk1_v, kg1_v, di1_v, ks1_v, s1_v, rows1_v, semi1, semg1)
        _finish(di0_v, s0_v, rows0_v, semg0)
        _fire_idx(g + 3, pk1_v, semi1)
        _prep(pk0_v, kg0_v, di0_v, ks0_v, s0_v, rows0_v, semi0, semg0)
        _finish(di1_v, s1_v, rows1_v, semg1)
        return 0

    lax.fori_loop(0, (NCH - 2) // 2, _pair, 0)
    _prep(pk1_v, kg1_v, di1_v, ks1_v, s1_v, rows1_v, semi1, semg1)
    _finish(di0_v, s0_v, rows0_v, semg0)
    _finish(di1_v, s1_v, rows1_v, semg1)
    plsc.subcore_barrier()

    for k in range(RPT // CH):
        pltpu.sync_copy(acc_sh.at[pl.ds(rbase + k * CH, CH)], rows0_v)
        pltpu.sync_copy(rows0_v, out_hbm.at[cid, pl.ds(rbase + k * CH, CH)])


def _fin_body(p_ref, b_ref, o_ref):
    o_ref[...] = p_ref[0] + p_ref[1] + b_ref[...]


def _combine(parts, h_bias):
    return pl.pallas_call(
        _fin_body,
        out_shape=jax.ShapeDtypeStruct((N_NODES, D), jnp.float32),
    )(parts, h_bias.reshape(1, D))


def kernel(x, edge_index, edge_type, W, h_bias):
    src = edge_index[0].astype(jnp.int32)
    dst = edge_index[1].astype(jnp.int32)
    et = edge_type.astype(jnp.int32)

    pad = E_PAD - N_EDGES
    src_p = jnp.pad(src, (0, pad))
    dst_p = jnp.pad(dst, (0, pad))
    et_p = jnp.pad(et, (0, pad), constant_values=N_REL)
    packed = src_p | (dst_p << 14) | (et_p << 28)

    xw = _xw_table(x, W)
    deg_parts = _deg_kernel(packed)
    inv = _inv_deg(deg_parts)
    parts = _edge_kernel(xw, packed, inv)[:, :N_NODES, :]
    return _combine(parts, h_bias)


# speedup vs baseline: 26.7474x; 1.0741x over previous
import functools

import jax
import jax.numpy as jnp
from jax import lax
from jax.experimental import pallas as pl
from jax.experimental.pallas import tpu as pltpu
from jax.experimental.pallas import tpu_sc as plsc

N_NODES = 10000
N_EDGES = 320000
D = 128
N_REL = 8

NTILES = 32
CH = 128
NCH0 = 120
NCH1 = 40
EPT0 = NCH0 * CH
EPT1 = NCH1 * CH
E_PAD = 16 * (EPT0 + EPT1)
NBINS = 81920
BPT = NBINS // 16
N_ROWS = 10240
RPT = N_ROWS // 16

_mesh = plsc.VectorSubcoreMesh(core_axis_name="c", subcore_axis_name="s")


def _mm_body(x_ref, w_ref, o_ref):
    o_ref[0] = jnp.dot(x_ref[...], w_ref[0],
                       preferred_element_type=jnp.float32)


def _xw_table(x, W):
    M_BLK = 2000
    xw = pl.pallas_call(
        _mm_body,
        grid=(N_REL, N_NODES // M_BLK),
        in_specs=[
            pl.BlockSpec((M_BLK, D), lambda r, m: (m, 0)),
            pl.BlockSpec((1, D, D), lambda r, m: (r, 0, 0)),
        ],
        out_specs=pl.BlockSpec((1, M_BLK, D), lambda r, m: (r, m, 0)),
        out_shape=jax.ShapeDtypeStruct((N_REL, N_NODES, D), jnp.float32),
    )(x, W)
    return xw.reshape(N_REL * N_NODES, D)


@functools.partial(
    pl.kernel,
    mesh=_mesh,
    out_type=jax.ShapeDtypeStruct((2 * NBINS,), jnp.float32),
    scratch_types=[
        pltpu.VMEM((EPT0,), jnp.int32),
        pltpu.VMEM((1, CH), jnp.int32),
        pltpu.VMEM((1, CH), jnp.int32),
        pltpu.VMEM((CH,), jnp.float32),
        pltpu.VMEM((BPT,), jnp.float32),
        pltpu.VMEM_SHARED((NBINS,), jnp.float32),
        pltpu.SemaphoreType.DMA,
        pltpu.SemaphoreType.DMA,
    ],
)
def _deg_kernel(pk_hbm, out_hbm, pk_v, k0_v, k1_v, ones_v,
                bounce_v, hist_sh, sem0, sem1):
    cid = lax.axis_index("c")
    sid = lax.axis_index("s")
    base = jnp.where(cid == 0, sid * EPT0, 16 * EPT0 + sid * EPT1)
    nch = jnp.where(cid == 0, NCH0, NCH1)

    def _zero16(i, _):
        bounce_v[pl.ds(i * 16, 16)] = jnp.zeros((16,), jnp.float32)
        return 0

    lax.fori_loop(0, BPT // 16, _zero16, 0)

    def _ones16(i, _):
        ones_v[pl.ds(i * 16, 16)] = jnp.ones((16,), jnp.float32)
        return 0

    lax.fori_loop(0, CH // 16, _ones16, 0)

    pltpu.sync_copy(pk_hbm.at[pl.ds(base, EPT1)], pk_v.at[pl.ds(0, EPT1)])

    @pl.when(cid == 0)
    def _():
        pltpu.sync_copy(pk_hbm.at[pl.ds(base + EPT1, EPT0 - EPT1)],
                        pk_v.at[pl.ds(EPT1, EPT0 - EPT1)])

    pltpu.sync_copy(bounce_v, hist_sh.at[pl.ds(sid * BPT, BPT)])
    plsc.subcore_barrier()

    def _keys(j, kv):
        def _k16(i, _):
            p = pk_v[pl.ds(j * CH + i * 16, 16)]
            t = lax.shift_right_logical(p, 28)
            d = jnp.bitwise_and(lax.shift_right_logical(p, 14), 16383)
            kv[0, pl.ds(i * 16, 16)] = t * N_NODES + d
            return 0
        lax.fori_loop(0, CH // 16, _k16, 0)

    def _fire(kv, sem):
        pltpu.async_copy(ones_v, hist_sh.at[kv.at[0]], sem, add=True)

    def _drain(sem):
        pltpu.make_async_copy(ones_v, hist_sh.at[k0_v.at[0]], sem).wait()

    _keys(0, k0_v)
    _fire(k0_v, sem0)
    _keys(1, k1_v)
    _fire(k1_v, sem1)

    def _pair(g2, _):
        g = g2 * 2
        _drain(sem0)
        _keys(g + 2, k0_v)
        _fire(k0_v, sem0)
        _drain(sem1)
        _keys(g + 3, k1_v)
        _fire(k1_v, sem1)
        return 0

    lax.fori_loop(0, (nch - 2) // 2, _pair, 0)
    _drain(sem0)
    _drain(sem1)
    plsc.subcore_barrier()

    pltpu.sync_copy(hist_sh.at[pl.ds(sid * BPT, BPT)], bounce_v)
    pltpu.sync_copy(bounce_v, out_hbm.at[pl.ds(cid * NBINS + sid * BPT, BPT)])


def _inv_body(p_ref, o_ref):
    s = p_ref[0] + p_ref[1]
    row = lax.broadcasted_iota(jnp.int32, (NBINS // 128, 128), 0)
    col = lax.broadcasted_iota(jnp.int32, (NBINS // 128, 128), 1)
    real = (row * 128 + col) < N_REL * N_NODES
    o_ref[...] = jnp.where(real, 1.0 / jnp.maximum(s, 1.0), 0.0)


def _inv_deg(parts):
    inv = pl.pallas_call(
        _inv_body,
        out_shape=jax.ShapeDtypeStruct((NBINS // 128, 128), jnp.float32),
    )(parts.reshape(2, NBINS // 128, 128))
    return inv.reshape(NBINS)


@functools.partial(
    pl.kernel,
    mesh=_mesh,
    out_type=jax.ShapeDtypeStruct((2, N_ROWS, D), jnp.float32),
    scratch_types=[
        pltpu.VMEM((CH,), jnp.int32),
        pltpu.VMEM((CH,), jnp.int32),
        pltpu.VMEM((1, CH), jnp.int32),
        pltpu.VMEM((1, CH), jnp.int32),
        pltpu.VMEM((1, CH), jnp.int32),
        pltpu.VMEM((1, CH), jnp.int32),
        pltpu.VMEM((1, CH), jnp.int32),
        pltpu.VMEM((1, CH), jnp.int32),
        pltpu.VMEM((CH,), jnp.float32),
        pltpu.VMEM((CH,), jnp.float32),
        pltpu.VMEM((CH, D), jnp.float32),
        pltpu.VMEM((CH, D), jnp.float32),
        pltpu.VMEM_SHARED((N_ROWS, D), jnp.float32),
        pltpu.SemaphoreType.DMA,
        pltpu.SemaphoreType.DMA,
        pltpu.SemaphoreType.DMA,
        pltpu.SemaphoreType.DMA,
    ],
)
def _edge_kernel(xw_hbm, pk_hbm, inv_hbm, out_hbm,
                 pk0_v, pk1_v, kg0_v, kg1_v, di0_v, di1_v, ks0_v, ks1_v,
                 s0_v, s1_v, rows0_v, rows1_v, acc_sh,
                 semi0, semi1, semg0, semg1):
    cid = lax.axis_index("c")
    sid = lax.axis_index("s")
    nch = jnp.where(cid == 0, NCH0, NCH1)

    def _zrow(i, _):
        for q in range(D // 16):
            rows0_v[i, pl.ds(q * 16, 16)] = jnp.zeros((16,), jnp.float32)
        return 0

    lax.fori_loop(0, CH, _zrow, 0)

    rbase = sid * RPT
    for k in range(RPT // CH):
        pltpu.sync_copy(rows0_v, acc_sh.at[pl.ds(rbase + k * CH, CH)])
    plsc.subcore_barrier()

    base = jnp.where(cid == 0, sid * EPT0, 16 * EPT0 + sid * EPT1)

    def _fire_idx(j, pk_v, sem):
        pltpu.async_copy(pk_hbm.at[pl.ds(base + j * CH, CH)], pk_v, sem)

    def _prep(pk_v, kg_v, di_v, ks_v, s_v, rows_v, semi, semg):
        pltpu.make_async_copy(pk_hbm.at[pl.ds(0, CH)], pk_v, semi).wait()

        def _k16(i, _):
            sl = pl.ds(i * 16, 16)
            p = pk_v[sl]
            t = lax.shift_right_logical(p, 28)
            d = jnp.bitwise_and(lax.shift_right_logical(p, 14), 16383)
            s16 = jnp.bitwise_and(p, 16383)
            valid = t < N_REL
            kg_v[0, sl] = jnp.where(valid, t * N_NODES + s16, 0)
            di_v[0, sl] = d
            ks_v[0, sl] = t * N_NODES + d
            return 0

        lax.fori_loop(0, CH // 16, _k16, 0)
        pltpu.async_copy(inv_hbm.at[ks_v.at[0]], s_v, semg)
        pltpu.async_copy(xw_hbm.at[kg_v.at[0]], rows_v, semg)

    def _finish(di_v, s_v, rows_v, semg):
        pltpu.make_async_copy(inv_hbm.at[pl.ds(0, CH)], s_v, semg).wait()
        pltpu.make_async_copy(xw_hbm.at[pl.ds(0, CH)], rows_v, semg).wait()

        def _sg(g, _):
            s16 = s_v[pl.ds(g * 16, 16)]
            for i16 in range(16):
                sc = lax.gather(
                    s16, jnp.full((16, 1), i16, dtype=jnp.int32),
                    lax.GatherDimensionNumbers(
                        offset_dims=(), collapsed_slice_dims=(0,),
                        start_index_map=(0,)),
                    (1,), mode=lax.GatherScatterMode.PROMISE_IN_BOUNDS)
                row = g * 16 + i16
                for q in range(D // 16):
                    sl = pl.ds(q * 16, 16)
                    rows_v[row, sl] = rows_v[row, sl] * sc
            return 0

        lax.fori_loop(0, CH // 16, _sg, 0)
        pltpu.sync_copy(rows_v, acc_sh.at[di_v.at[0]], add=True)

    _fire_idx(0, pk0_v, semi0)
    _fire_idx(1, pk1_v, semi1)
    _prep(pk0_v, kg0_v, di0_v, ks0_v, s0_v, rows0_v, semi0, semg0)

    def _pair(g2, _):
        g = g2 * 2
        _fire_idx(g + 2, pk0_v, semi0)
        _prep(pk1_v, kg1_v, di1_v, ks1_v, s1_v, rows1_v, semi1, semg1)
        _finish(di0_v, s0_v, rows0_v, semg0)
        _fire_idx(g + 3, pk1_v, semi1)
        _prep(pk0_v, kg0_v, di0_v, ks0_v, s0_v, rows0_v, semi0, semg0)
        _finish(di1_v, s1_v, rows1_v, semg1)
        return 0

    lax.fori_loop(0, (nch - 2) // 2, _pair, 0)
    _prep(pk1_v, kg1_v, di1_v, ks1_v, s1_v, rows1_v, semi1, semg1)
    _finish(di0_v, s0_v, rows0_v, semg0)
    _finish(di1_v, s1_v, rows1_v, semg1)
    plsc.subcore_barrier()

    for k in range(RPT // CH):
        pltpu.sync_copy(acc_sh.at[pl.ds(rbase + k * CH, CH)], rows0_v)
        pltpu.sync_copy(rows0_v, out_hbm.at[cid, pl.ds(rbase + k * CH, CH)])


def _fin_body(p_ref, b_ref, o_ref):
    o_ref[...] = p_ref[0] + p_ref[1] + b_ref[...]


def _combine(parts, h_bias):
    return pl.pallas_call(
        _fin_body,
        out_shape=jax.ShapeDtypeStruct((N_NODES, D), jnp.float32),
    )(parts, h_bias.reshape(1, D))


def kernel(x, edge_index, edge_type, W, h_bias):
    src = edge_index[0].astype(jnp.int32)
    dst = edge_index[1].astype(jnp.int32)
    et = edge_type.astype(jnp.int32)

    pad = E_PAD - N_EDGES
    src_p = jnp.pad(src, (0, pad))
    dst_p = jnp.pad(dst, (0, pad))
    et_p = jnp.pad(et, (0, pad), constant_values=N_REL)
    packed = src_p | (dst_p << 14) | (et_p << 28)

    xw = _xw_table(x, W)
    deg_parts = _deg_kernel(packed)
    inv = _inv_deg(deg_parts)
    parts = _edge_kernel(xw, packed, inv)[:, :N_NODES, :]
    return _combine(parts, h_bias)


# speedup vs baseline: 27.5096x; 1.0285x over previous
import functools

import jax
import jax.numpy as jnp
from jax import lax
from jax.experimental import pallas as pl
from jax.experimental.pallas import tpu as pltpu
from jax.experimental.pallas import tpu_sc as plsc

N_NODES = 10000
N_EDGES = 320000
D = 128
N_REL = 8

NTILES = 32
CH = 128
NCH0 = 140
NCH1 = 20
EPT0 = NCH0 * CH
EPT1 = NCH1 * CH
E_PAD = 16 * (EPT0 + EPT1)
NBINS = 81920
BPT = NBINS // 16
N_ROWS = 10240
RPT = N_ROWS // 16

_mesh = plsc.VectorSubcoreMesh(core_axis_name="c", subcore_axis_name="s")


def _mm_body(x_ref, w_ref, o_ref):
    o_ref[0] = jnp.dot(x_ref[...], w_ref[0],
                       preferred_element_type=jnp.float32)


def _xw_table(x, W):
    M_BLK = 2000
    xw = pl.pallas_call(
        _mm_body,
        grid=(N_REL, N_NODES // M_BLK),
        in_specs=[
            pl.BlockSpec((M_BLK, D), lambda r, m: (m, 0)),
            pl.BlockSpec((1, D, D), lambda r, m: (r, 0, 0)),
        ],
        out_specs=pl.BlockSpec((1, M_BLK, D), lambda r, m: (r, m, 0)),
        out_shape=jax.ShapeDtypeStruct((N_REL, N_NODES, D), jnp.float32),
    )(x, W)
    return xw.reshape(N_REL * N_NODES, D)


@functools.partial(
    pl.kernel,
    mesh=_mesh,
    out_type=jax.ShapeDtypeStruct((2 * NBINS,), jnp.float32),
    scratch_types=[
        pltpu.VMEM((EPT0,), jnp.int32),
        pltpu.VMEM((1, CH), jnp.int32),
        pltpu.VMEM((1, CH), jnp.int32),
        pltpu.VMEM((CH,), jnp.float32),
        pltpu.VMEM((BPT,), jnp.float32),
        pltpu.VMEM_SHARED((NBINS,), jnp.float32),
        pltpu.SemaphoreType.DMA,
        pltpu.SemaphoreType.DMA,
    ],
)
def _deg_kernel(pk_hbm, out_hbm, pk_v, k0_v, k1_v, ones_v,
                bounce_v, hist_sh, sem0, sem1):
    cid = lax.axis_index("c")
    sid = lax.axis_index("s")
    base = jnp.where(cid == 0, sid * EPT0, 16 * EPT0 + sid * EPT1)
    nch = jnp.where(cid == 0, NCH0, NCH1)

    def _zero16(i, _):
        bounce_v[pl.ds(i * 16, 16)] = jnp.zeros((16,), jnp.float32)
        return 0

    lax.fori_loop(0, BPT // 16, _zero16, 0)

    def _ones16(i, _):
        ones_v[pl.ds(i * 16, 16)] = jnp.ones((16,), jnp.float32)
        return 0

    lax.fori_loop(0, CH // 16, _ones16, 0)

    pltpu.sync_copy(pk_hbm.at[pl.ds(base, EPT1)], pk_v.at[pl.ds(0, EPT1)])

    @pl.when(cid == 0)
    def _():
        pltpu.sync_copy(pk_hbm.at[pl.ds(base + EPT1, EPT0 - EPT1)],
                        pk_v.at[pl.ds(EPT1, EPT0 - EPT1)])

    pltpu.sync_copy(bounce_v, hist_sh.at[pl.ds(sid * BPT, BPT)])
    plsc.subcore_barrier()

    def _keys(j, kv):
        def _k16(i, _):
            p = pk_v[pl.ds(j * CH + i * 16, 16)]
            t = lax.shift_right_logical(p, 28)
            d = jnp.bitwise_and(lax.shift_right_logical(p, 14), 16383)
            kv[0, pl.ds(i * 16, 16)] = t * N_NODES + d
            return 0
        lax.fori_loop(0, CH // 16, _k16, 0)

    def _fire(kv, sem):
        pltpu.async_copy(ones_v, hist_sh.at[kv.at[0]], sem, add=True)

    def _drain(sem):
        pltpu.make_async_copy(ones_v, hist_sh.at[k0_v.at[0]], sem).wait()

    _keys(0, k0_v)
    _fire(k0_v, sem0)
    _keys(1, k1_v)
    _fire(k1_v, sem1)

    def _pair(g2, _):
        g = g2 * 2
        _drain(sem0)
        _keys(g + 2, k0_v)
        _fire(k0_v, sem0)
        _drain(sem1)
        _keys(g + 3, k1_v)
        _fire(k1_v, sem1)
        return 0

    lax.fori_loop(0, (nch - 2) // 2, _pair, 0)
    _drain(sem0)
    _drain(sem1)
    plsc.subcore_barrier()

    pltpu.sync_copy(hist_sh.at[pl.ds(sid * BPT, BPT)], bounce_v)
    pltpu.sync_copy(bounce_v, out_hbm.at[pl.ds(cid * NBINS + sid * BPT, BPT)])


def _inv_body(p_ref, o_ref):
    s = p_ref[0] + p_ref[1]
    row = lax.broadcasted_iota(jnp.int32, (NBINS // 128, 128), 0)
    col = lax.broadcasted_iota(jnp.int32, (NBINS // 128, 128), 1)
    real = (row * 128 + col) < N_REL * N_NODES
    o_ref[...] = jnp.where(real, 1.0 / jnp.maximum(s, 1.0), 0.0)


def _inv_deg(parts):
    inv = pl.pallas_call(
        _inv_body,
        out_shape=jax.ShapeDtypeStruct((NBINS // 128, 128), jnp.float32),
    )(parts.reshape(2, NBINS // 128, 128))
    return inv.reshape(NBINS)


@functools.partial(
    pl.kernel,
    mesh=_mesh,
    out_type=jax.ShapeDtypeStruct((2, N_ROWS, D), jnp.float32),
    scratch_types=[
        pltpu.VMEM((CH,), jnp.int32),
        pltpu.VMEM((CH,), jnp.int32),
        pltpu.VMEM((1, CH), jnp.int32),
        pltpu.VMEM((1, CH), jnp.int32),
        pltpu.VMEM((1, CH), jnp.int32),
        pltpu.VMEM((1, CH), jnp.int32),
        pltpu.VMEM((1, CH), jnp.int32),
        pltpu.VMEM((1, CH), jnp.int32),
        pltpu.VMEM((CH,), jnp.float32),
        pltpu.VMEM((CH,), jnp.float32),
        pltpu.VMEM((CH, D), jnp.float32),
        pltpu.VMEM((CH, D), jnp.float32),
        pltpu.VMEM_SHARED((N_ROWS, D), jnp.float32),
        pltpu.SemaphoreType.DMA,
        pltpu.SemaphoreType.DMA,
        pltpu.SemaphoreType.DMA,
        pltpu.SemaphoreType.DMA,
    ],
)
def _edge_kernel(xw_hbm, pk_hbm, inv_hbm, out_hbm,
                 pk0_v, pk1_v, kg0_v, kg1_v, di0_v, di1_v, ks0_v, ks1_v,
                 s0_v, s1_v, rows0_v, rows1_v, acc_sh,
                 semi0, semi1, semg0, semg1):
    cid = lax.axis_index("c")
    sid = lax.axis_index("s")
    nch = jnp.where(cid == 0, NCH0, NCH1)

    def _zrow(i, _):
        for q in range(D // 16):
            rows0_v[i, pl.ds(q * 16, 16)] = jnp.zeros((16,), jnp.float32)
        return 0

    lax.fori_loop(0, CH, _zrow, 0)

    rbase = sid * RPT
    for k in range(RPT // CH):
        pltpu.sync_copy(rows0_v, acc_sh.at[pl.ds(rbase + k * CH, CH)])

    plsc.subcore_barrier()

    base = jnp.where(cid == 0, sid * EPT0, 16 * EPT0 + sid * EPT1)

    def _fire_idx(j, pk_v, sem):
        pltpu.async_copy(pk_hbm.at[pl.ds(base + j * CH, CH)], pk_v, sem)

    def _prep(pk_v, kg_v, di_v, ks_v, s_v, rows_v, semi, semg):
        pltpu.make_async_copy(pk_hbm.at[pl.ds(0, CH)], pk_v, semi).wait()

        def _k16(i, _):
            sl = pl.ds(i * 16, 16)
            p = pk_v[sl]
            t = lax.shift_right_logical(p, 28)
            d = jnp.bitwise_and(lax.shift_right_logical(p, 14), 16383)
            s16 = jnp.bitwise_and(p, 16383)
            valid = t < N_REL
            kg_v[0, sl] = jnp.where(valid, t * N_NODES + s16, 0)
            di_v[0, sl] = d
            ks_v[0, sl] = t * N_NODES + d
            return 0

        lax.fori_loop(0, CH // 16, _k16, 0)
        pltpu.async_copy(inv_hbm.at[ks_v.at[0]], s_v, semg)
        pltpu.async_copy(xw_hbm.at[kg_v.at[0]], rows_v, semg)

    def _finish(di_v, s_v, rows_v, semg):
        pltpu.make_async_copy(inv_hbm.at[pl.ds(0, CH)], s_v, semg).wait()
        pltpu.make_async_copy(xw_hbm.at[pl.ds(0, CH)], rows_v, semg).wait()

        def _sg(g, _):
            s16 = s_v[pl.ds(g * 16, 16)]
            for i16 in range(16):
                sc = lax.gather(
                    s16, jnp.full((16, 1), i16, dtype=jnp.int32),
                    lax.GatherDimensionNumbers(
                        offset_dims=(), collapsed_slice_dims=(0,),
                        start_index_map=(0,)),
                    (1,), mode=lax.GatherScatterMode.PROMISE_IN_BOUNDS)
                row = g * 16 + i16
                for q in range(D // 16):
                    sl = pl.ds(q * 16, 16)
                    rows_v[row, sl] = rows_v[row, sl] * sc
            return 0

        lax.fori_loop(0, CH // 16, _sg, 0)
        pltpu.sync_copy(rows_v, acc_sh.at[di_v.at[0]], add=True)

    _fire_idx(0, pk0_v, semi0)
    _fire_idx(1, pk1_v, semi1)
    _prep(pk0_v, kg0_v, di0_v, ks0_v, s0_v, rows0_v, semi0, semg0)

    def _pair(g2, _):
        g = g2 * 2
        _fire_idx(g + 2, pk0_v, semi0)
        _prep(pk1_v, kg1_v, di1_v, ks1_v, s1_v, rows1_v, semi1, semg1)
        _finish(di0_v, s0_v, rows0_v, semg0)
        _fire_idx(g + 3, pk1_v, semi1)
        _prep(pk0_v, kg0_v, di0_v, ks0_v, s0_v, rows0_v, semi0, semg0)
        _finish(di1_v, s1_v, rows1_v, semg1)
        return 0

    lax.fori_loop(0, (nch - 2) // 2, _pair, 0)
    _prep(pk1_v, kg1_v, di1_v, ks1_v, s1_v, rows1_v, semi1, semg1)
    _finish(di0_v, s0_v, rows0_v, semg0)
    _finish(di1_v, s1_v, rows1_v, semg1)
    plsc.subcore_barrier()

    for k in range(RPT // CH):
        pltpu.sync_copy(acc_sh.at[pl.ds(rbase + k * CH, CH)], rows0_v)
        pltpu.sync_copy(rows0_v, out_hbm.at[cid, pl.ds(rbase + k * CH, CH)])


def _fin_body(p_ref, b_ref, o_ref):
    o_ref[...] = p_ref[0] + p_ref[1] + b_ref[...]


def _combine(parts, h_bias):
    return pl.pallas_call(
        _fin_body,
        out_shape=jax.ShapeDtypeStruct((N_NODES, D), jnp.float32),
    )(parts, h_bias.reshape(1, D))


def kernel(x, edge_index, edge_type, W, h_bias):
    src = edge_index[0].astype(jnp.int32)
    dst = edge_index[1].astype(jnp.int32)
    et = edge_type.astype(jnp.int32)

    pad = E_PAD - N_EDGES
    src_p = jnp.pad(src, (0, pad))
    dst_p = jnp.pad(dst, (0, pad))
    et_p = jnp.pad(et, (0, pad), constant_values=N_REL)
    packed = src_p | (dst_p << 14) | (et_p << 28)

    xw = _xw_table(x, W)
    deg_parts = _deg_kernel(packed)
    inv = _inv_deg(deg_parts)
    parts = _edge_kernel(xw, packed, inv)[:, :N_NODES, :]
    return _combine(parts, h_bias)
